# Initial kernel scaffold; baseline (speedup 1.0000x reference)
import jax
import jax.numpy as jnp
from jax.experimental import pallas as pl


def kernel(x, edge_index, edge_weight, W_in, b_in, W1, b1, W2, b2, W3, b3, W_out, b_out):
    raise NotImplementedError("write your pallas kernel here")


# speedup vs baseline: 7.3032x; 7.3032x over previous
import functools

import jax
import jax.numpy as jnp
from jax import lax
from jax.experimental import pallas as pl
from jax.experimental.pallas import tpu as pltpu
from jax.experimental.pallas import tpu_sc as plsc

N = 10000
E = 320000
H = 128
C = 40

NC = 2
NS = 16
NW = NC * NS
L = 16

BLK = 128
NB_C0 = 58
NB_C1 = 100
NBMAX = max(NB_C0, NB_C1)
EWMAX = NBMAX * BLK
E_PAD = NS * (NB_C0 + NB_C1) * BLK
RPS = 624
NP = 10112

_f32 = jnp.float32
_i32 = jnp.int32
_bf16 = jnp.bfloat16

_mesh = plsc.VectorSubcoreMesh(
    core_axis_name="c", subcore_axis_name="s", num_cores=NC, num_subcores=NS)


def _wid():
    c = lax.axis_index("c")
    s = lax.axis_index("s")
    return c, s, c * NS + s


_SC_DEG_PARAMS = dict(
    out_type=jax.ShapeDtypeStruct((NW, NP), _f32),
    mesh=_mesh,
    compiler_params=pltpu.CompilerParams(needs_layout_passes=False),
    scratch_types=[
        pltpu.VMEM((EWMAX,), _i32),
        pltpu.VMEM((EWMAX,), _f32),
        pltpu.VMEM((NP,), _f32),
    ],
)


def _sc_deg_body(dstp_hbm, ewp_hbm, out_hbm, dstv, ewv, accv):
    _, _, w = _wid()
    pltpu.sync_copy(dstp_hbm.at[w], dstv)
    pltpu.sync_copy(ewp_hbm.at[w], ewv)

    def zbody(i, carry):
        accv[pl.ds(i * L, L)] = jnp.zeros((L,), _f32)
        return carry

    lax.fori_loop(0, NP // L, zbody, 0)

    def body(i, carry):
        sl = pl.ds(i * L, L)
        plsc.addupdate_scatter(accv, [dstv[sl]], ewv[sl])
        return carry

    lax.fori_loop(0, EWMAX // L, body, 0)
    pltpu.sync_copy(accv, out_hbm.at[w])


def _rsqrt16(d):
    xi = plsc.bitcast(d, _i32)
    yi = jnp.int32(0x5F3759DF) - lax.shift_right_logical(xi, 1)
    y = plsc.bitcast(yi, _f32)
    for _ in range(3):
        y = y * (1.5 - 0.5 * d * y * y)
    return y


_SC_NORM_PARAMS = dict(
    out_type=jax.ShapeDtypeStruct((NW, EWMAX // 2), _i32),
    mesh=_mesh,
    compiler_params=pltpu.CompilerParams(needs_layout_passes=False),
    scratch_types=[
        pltpu.VMEM((EWMAX,), _i32),
        pltpu.VMEM((EWMAX,), _i32),
        pltpu.VMEM((EWMAX,), _f32),
        pltpu.VMEM((NP,), _f32),
        pltpu.VMEM((NP,), _f32),
        pltpu.VMEM((EWMAX // 2,), _i32),
    ],
)


def _sc_norm_body(srcp_hbm, dstp_hbm, ewp_hbm, degp_hbm, out_hbm,
                  srcv, dstv, ewv, disv, pbuf, normh):
    _, _, w = _wid()
    pltpu.sync_copy(srcp_hbm.at[w], srcv)
    pltpu.sync_copy(dstp_hbm.at[w], dstv)
    pltpu.sync_copy(ewp_hbm.at[w], ewv)

    def zbody(i, carry):
        disv[pl.ds(i * L, L)] = jnp.zeros((L,), _f32)
        return carry

    lax.fori_loop(0, NP // L, zbody, 0)

    def abody(i, carry):
        sl = pl.ds(i * L, L)
        disv[sl] = disv[sl] + pbuf[sl]
        return carry

    for r in range(NW):
        pltpu.sync_copy(degp_hbm.at[r], pbuf)
        lax.fori_loop(0, NP // L, abody, 0)

    def dbody(i, carry):
        sl = pl.ds(i * L, L)
        disv[sl] = _rsqrt16(disv[sl] + 1.0)
        return carry

    lax.fori_loop(0, NP // L, dbody, 0)

    def body(g, carry):
        ns = []
        for half in range(2):
            sl = pl.ds(g * 2 * L + half * L, L)
            a = plsc.load_gather(disv, [srcv[sl]])
            b = plsc.load_gather(disv, [dstv[sl]])
            ns.append(a * ewv[sl] * b)
        packed = plsc.pack(ns[0], ns[1], format=plsc.PackFormat.INTERLEAVED)
        normh[pl.ds(g * L, L)] = plsc.bitcast(packed, _i32)
        return carry

    lax.fori_loop(0, EWMAX // (2 * L), body, 0)
    pltpu.sync_copy(normh, out_hbm.at[w])


_SC_AGG_PARAMS = dict(
    out_type=jax.ShapeDtypeStruct((NC, N, H), _f32),
    mesh=_mesh,
    compiler_params=pltpu.CompilerParams(needs_layout_passes=False),
    scratch_types=[
        pltpu.VMEM((EWMAX,), _i32),
        pltpu.VMEM((EWMAX,), _i32),
        pltpu.VMEM((EWMAX // 2,), _i32),
        pltpu.VMEM((BLK, H), _f32),
        pltpu.SemaphoreType.DMA,
        pltpu.SemaphoreType.DMA,
        pltpu.VMEM_SHARED((N, H), _f32),
    ],
)


def _sc_agg_body(hw_hbm, srcp_hbm, dstp_hbm, normp_hbm, zrows_hbm, out_hbm,
                 srcv, dstv, normh, gbuf, gsem, ssem, acc):
    c, s, w = _wid()
    pltpu.sync_copy(srcp_hbm.at[w], srcv)
    pltpu.sync_copy(dstp_hbm.at[w], dstv)
    pltpu.sync_copy(normp_hbm.at[w], normh)
    pltpu.sync_copy(zrows_hbm, acc.at[pl.ds(s * RPS, RPS)])

    @pl.when(s == NS - 1)
    def _():
        pltpu.sync_copy(zrows_hbm.at[pl.ds(0, N - NS * RPS)],
                        acc.at[pl.ds(NS * RPS, N - NS * RPS)])

    plsc.subcore_barrier()

    nblocks = jnp.where(c == 0, NB_C0, NB_C1)

    def blk_body(bi, carry):
        idx = srcv.at[pl.ds(bi * BLK, BLK)]
        pltpu.async_copy(hw_hbm.at[idx], gbuf, gsem).wait()

        def g_body(g, carry2):
            nh32 = normh[pl.ds(bi * (BLK // 2) + g * L, L)]
            nh = plsc.bitcast(nh32, _bf16)
            na, nb = plsc.unpack(nh, format=plsc.PackFormat.INTERLEAVED)
            for half, nv in ((0, na), (1, nb)):
                for k in range(L):
                    n = nv[k]
                    e = g * 2 * L + half * L + k
                    for j in range(H // L):
                        sl = pl.ds(j * L, L)
                        gbuf[e, sl] = gbuf[e, sl] * n
            return carry2

        lax.fori_loop(0, BLK // (2 * L), g_body, 0)
        didx = dstv.at[pl.ds(bi * BLK, BLK)]
        pltpu.async_copy(gbuf, acc.at[didx], ssem, add=True).wait()
        return carry

    lax.fori_loop(0, nblocks, blk_body, 0)
    plsc.subcore_barrier()
    sl = pl.ds(s * RPS, RPS)
    pltpu.sync_copy(acc.at[sl], out_hbm.at[c, sl])

    @pl.when(s == NS - 1)
    def _():
        sl2 = pl.ds(NS * RPS, N - NS * RPS)
        pltpu.sync_copy(acc.at[sl2], out_hbm.at[c, sl2])


_sc_deg = pl.kernel(_sc_deg_body, **_SC_DEG_PARAMS)
_sc_norm = pl.kernel(_sc_norm_body, **_SC_NORM_PARAMS)
_sc_agg = pl.kernel(_sc_agg_body, **_SC_AGG_PARAMS)


_CN = (((1,), (1,)), ((), ()))


def _tc_in_body(x_ref, wi_ref, bi_ref, w1_ref, degp_ref,
                hw1_ref, invdeg_ref):
    h = lax.dot_general(x_ref[...], wi_ref[...], _CN,
                        preferred_element_type=_f32)
    h = jnp.maximum(h + bi_ref[...][None, :], 0.0)
    hw1_ref[...] = lax.dot_general(h, w1_ref[...], _CN,
                                   preferred_element_type=_f32)
    ones = jnp.ones((NW, 1), _f32)
    deg2 = lax.dot_general(degp_ref[...], ones, (((0,), (0,)), ((), ())),
                           preferred_element_type=_f32) + 1.0
    invdeg_ref[...] = 1.0 / deg2[:N, :]


def _tc_in(x, w_in, b_in, w1, degp):
    return pl.pallas_call(
        _tc_in_body,
        out_shape=(
            jax.ShapeDtypeStruct((N, H), _f32),
            jax.ShapeDtypeStruct((N, 1), _f32),
        ),
    )(x, w_in, b_in, w1, degp)


def _tc_mid_body(agg_ref, hwp_ref, invdeg_ref, b_ref, w_ref, out_ref):
    acc = (agg_ref[0] + agg_ref[1]
           + invdeg_ref[...] * hwp_ref[...] + b_ref[...][None, :])
    h = jnp.maximum(acc, 0.0)
    out_ref[...] = lax.dot_general(h, w_ref[...], _CN,
                                   preferred_element_type=_f32)


def _tc_mid(agg, hw_prev, invdeg, b, w_next):
    return pl.pallas_call(
        _tc_mid_body,
        out_shape=jax.ShapeDtypeStruct((N, H), _f32),
    )(agg, hw_prev, invdeg, b, w_next)


def _tc_out_body(agg_ref, hwp_ref, invdeg_ref, b_ref, wo_ref, bo_ref, out_ref):
    acc = (agg_ref[0] + agg_ref[1]
           + invdeg_ref[...] * hwp_ref[...] + b_ref[...][None, :])
    h = jnp.maximum(acc, 0.0)
    out_ref[...] = lax.dot_general(h, wo_ref[...], _CN,
                                   preferred_element_type=_f32) \
        + bo_ref[...][None, :]


def _tc_out(agg, hw_prev, invdeg, b, w_out, b_out):
    return pl.pallas_call(
        _tc_out_body,
        out_shape=jax.ShapeDtypeStruct((N, C), _f32),
    )(agg, hw_prev, invdeg, b, w_out, b_out)


def _layout(a, pad_value):
    n0 = NS * NB_C0 * BLK
    p0 = a[:n0].reshape(NS, NB_C0 * BLK)
    p0 = jnp.pad(p0, ((0, 0), (0, EWMAX - NB_C0 * BLK)),
                 constant_values=pad_value)
    p1 = a[n0:].reshape(NS, NB_C1 * BLK)
    p1 = jnp.pad(p1, ((0, 0), (0, EWMAX - NB_C1 * BLK)),
                 constant_values=pad_value)
    return jnp.concatenate([p0, p1], axis=0)


def kernel(x, edge_index, edge_weight, W_in, b_in, W1, b1, W2, b2, W3, b3,
           W_out, b_out):
    src = edge_index[0]
    dst = edge_index[1]
    pad = E_PAD - E
    zpad_i = jnp.zeros((pad,), _i32)
    srcp = _layout(jnp.concatenate([src, zpad_i]), 0)
    dstp = _layout(jnp.concatenate([dst, zpad_i]), 0)
    ewp = _layout(jnp.concatenate([edge_weight, jnp.zeros((pad,), _f32)]), 0)
    zrows = jnp.zeros((RPS, H), _f32)

    degp = _sc_deg(dstp, ewp)
    hw1, invdeg = _tc_in(x, W_in, b_in, W1, degp)
    normp = _sc_norm(srcp, dstp, ewp, degp)
    agg1 = _sc_agg(hw1, srcp, dstp, normp, zrows)
    hw2 = _tc_mid(agg1, hw1, invdeg, b1, W2)
    agg2 = _sc_agg(hw2, srcp, dstp, normp, zrows)
    hw3 = _tc_mid(agg2, hw2, invdeg, b2, W3)
    agg3 = _sc_agg(hw3, srcp, dstp, normp, zrows)
    return _tc_out(agg3, hw3, invdeg, b3, W_out, b_out)


# speedup vs baseline: 9.6834x; 1.3259x over previous
import functools

import jax
import jax.numpy as jnp
from jax import lax
from jax.experimental import pallas as pl
from jax.experimental.pallas import tpu as pltpu
from jax.experimental.pallas import tpu_sc as plsc

N = 10000
E = 320000
H = 128
C = 40

NC = 2
NS = 16
NW = NC * NS
L = 16

BLK = 128
NB_C0 = 97
NB_C1 = 61
NBMAX = max(NB_C0, NB_C1)
EWMAX = NBMAX * BLK
E_PAD = NS * (NB_C0 + NB_C1) * BLK
RPS = 624
NP = 10112

_f32 = jnp.float32
_i32 = jnp.int32
_bf16 = jnp.bfloat16

_mesh = plsc.VectorSubcoreMesh(
    core_axis_name="c", subcore_axis_name="s", num_cores=NC, num_subcores=NS)


def _wid():
    c = lax.axis_index("c")
    s = lax.axis_index("s")
    return c, s, c * NS + s


_SC_DEG_PARAMS = dict(
    out_type=jax.ShapeDtypeStruct((NW, NP), _f32),
    mesh=_mesh,
    compiler_params=pltpu.CompilerParams(needs_layout_passes=False),
    scratch_types=[
        pltpu.VMEM((EWMAX,), _i32),
        pltpu.VMEM((EWMAX,), _f32),
        pltpu.VMEM((NP,), _f32),
    ],
)


def _sc_deg_body(dstp_hbm, ewp_hbm, out_hbm, dstv, ewv, accv):
    _, _, w = _wid()
    pltpu.sync_copy(dstp_hbm.at[w], dstv)
    pltpu.sync_copy(ewp_hbm.at[w], ewv)

    def zbody(i, carry):
        accv[pl.ds(i * L, L)] = jnp.zeros((L,), _f32)
        return carry

    lax.fori_loop(0, NP // L, zbody, 0)

    def body(i, carry):
        sl = pl.ds(i * L, L)
        plsc.addupdate_scatter(accv, [dstv[sl]], ewv[sl])
        return carry

    lax.fori_loop(0, EWMAX // L, body, 0)
    pltpu.sync_copy(accv, out_hbm.at[w])


def _rsqrt16(d):
    xi = plsc.bitcast(d, _i32)
    yi = jnp.int32(0x5F3759DF) - lax.shift_right_logical(xi, 1)
    y = plsc.bitcast(yi, _f32)
    for _ in range(3):
        y = y * (1.5 - 0.5 * d * y * y)
    return y


_SC_NORM_PARAMS = dict(
    out_type=jax.ShapeDtypeStruct((NW, EWMAX // 2), _i32),
    mesh=_mesh,
    compiler_params=pltpu.CompilerParams(needs_layout_passes=False),
    scratch_types=[
        pltpu.VMEM((EWMAX,), _i32),
        pltpu.VMEM((EWMAX,), _i32),
        pltpu.VMEM((EWMAX,), _f32),
        pltpu.VMEM((NP,), _f32),
        pltpu.VMEM((NP,), _f32),
        pltpu.VMEM((EWMAX // 2,), _i32),
    ],
)


def _sc_norm_body(srcp_hbm, dstp_hbm, ewp_hbm, degp_hbm, out_hbm,
                  srcv, dstv, ewv, disv, pbuf, normh):
    _, _, w = _wid()
    pltpu.sync_copy(srcp_hbm.at[w], srcv)
    pltpu.sync_copy(dstp_hbm.at[w], dstv)
    pltpu.sync_copy(ewp_hbm.at[w], ewv)

    def zbody(i, carry):
        disv[pl.ds(i * L, L)] = jnp.zeros((L,), _f32)
        return carry

    lax.fori_loop(0, NP // L, zbody, 0)

    def abody(i, carry):
        sl = pl.ds(i * L, L)
        disv[sl] = disv[sl] + pbuf[sl]
        return carry

    for r in range(NW):
        pltpu.sync_copy(degp_hbm.at[r], pbuf)
        lax.fori_loop(0, NP // L, abody, 0)

    def dbody(i, carry):
        sl = pl.ds(i * L, L)
        disv[sl] = _rsqrt16(disv[sl] + 1.0)
        return carry

    lax.fori_loop(0, NP // L, dbody, 0)

    def body(g, carry):
        ns = []
        for half in range(2):
            sl = pl.ds(g * 2 * L + half * L, L)
            a = plsc.load_gather(disv, [srcv[sl]])
            b = plsc.load_gather(disv, [dstv[sl]])
            ns.append(a * ewv[sl] * b)
        packed = plsc.pack(ns[0], ns[1], format=plsc.PackFormat.INTERLEAVED)
        normh[pl.ds(g * L, L)] = plsc.bitcast(packed, _i32)
        return carry

    lax.fori_loop(0, EWMAX // (2 * L), body, 0)
    pltpu.sync_copy(normh, out_hbm.at[w])


_SC_AGG_PARAMS = dict(
    out_type=jax.ShapeDtypeStruct((NC, N, H), _f32),
    mesh=_mesh,
    compiler_params=pltpu.CompilerParams(needs_layout_passes=False),
    scratch_types=[
        pltpu.VMEM((EWMAX,), _i32),
        pltpu.VMEM((EWMAX,), _i32),
        pltpu.VMEM((EWMAX // 2,), _i32),
        pltpu.VMEM((BLK, H), _f32),
        pltpu.SemaphoreType.DMA,
        pltpu.SemaphoreType.DMA,
        pltpu.VMEM_SHARED((N, H), _f32),
    ],
)


def _sc_agg_body(hw_hbm, srcp_hbm, dstp_hbm, normp_hbm, zrows_hbm, out_hbm,
                 srcv, dstv, normh, gbuf, gsem, ssem, acc):
    c, s, w = _wid()
    pltpu.sync_copy(srcp_hbm.at[w], srcv)
    pltpu.sync_copy(dstp_hbm.at[w], dstv)
    pltpu.sync_copy(normp_hbm.at[w], normh)
    pltpu.sync_copy(zrows_hbm, acc.at[pl.ds(s * RPS, RPS)])

    @pl.when(s == NS - 1)
    def _():
        pltpu.sync_copy(zrows_hbm.at[pl.ds(0, N - NS * RPS)],
                        acc.at[pl.ds(NS * RPS, N - NS * RPS)])

    plsc.subcore_barrier()

    nblocks = jnp.where(c == 0, NB_C0, NB_C1)

    def blk_body(bi, carry):
        idx = srcv.at[pl.ds(bi * BLK, BLK)]
        pltpu.async_copy(hw_hbm.at[idx], gbuf, gsem).wait()

        def g_body(g, carry2):
            nh32 = normh[pl.ds(bi * (BLK // 2) + g * L, L)]
            nh = plsc.bitcast(nh32, _bf16)
            na, nb = plsc.unpack(nh, format=plsc.PackFormat.INTERLEAVED)
            for half, nv in ((0, na), (1, nb)):
                for k in range(L):
                    n = nv[k]
                    e = g * 2 * L + half * L + k
                    for j in range(H // L):
                        sl = pl.ds(j * L, L)
                        gbuf[e, sl] = gbuf[e, sl] * n
            return carry2

        lax.fori_loop(0, BLK // (2 * L), g_body, 0)
        didx = dstv.at[pl.ds(bi * BLK, BLK)]
        pltpu.async_copy(gbuf, acc.at[didx], ssem, add=True).wait()
        return carry

    lax.fori_loop(0, nblocks, blk_body, 0)
    plsc.subcore_barrier()
    sl = pl.ds(s * RPS, RPS)
    pltpu.sync_copy(acc.at[sl], out_hbm.at[c, sl])

    @pl.when(s == NS - 1)
    def _():
        sl2 = pl.ds(NS * RPS, N - NS * RPS)
        pltpu.sync_copy(acc.at[sl2], out_hbm.at[c, sl2])


_sc_deg = pl.kernel(_sc_deg_body, **_SC_DEG_PARAMS)
_sc_norm = pl.kernel(_sc_norm_body, **_SC_NORM_PARAMS)
_sc_agg = pl.kernel(_sc_agg_body, **_SC_AGG_PARAMS)


_CN = (((1,), (1,)), ((), ()))


def _tc_in_body(x_ref, wi_ref, bi_ref, w1_ref, degp_ref,
                hw1_ref, invdeg_ref):
    h = lax.dot_general(x_ref[...], wi_ref[...], _CN,
                        preferred_element_type=_f32)
    h = jnp.maximum(h + bi_ref[...][None, :], 0.0)
    hw1_ref[...] = lax.dot_general(h, w1_ref[...], _CN,
                                   preferred_element_type=_f32)
    ones = jnp.ones((NW, 1), _f32)
    deg2 = lax.dot_general(degp_ref[...], ones, (((0,), (0,)), ((), ())),
                           preferred_element_type=_f32) + 1.0
    invdeg_ref[...] = 1.0 / deg2[:N, :]


def _tc_in(x, w_in, b_in, w1, degp):
    return pl.pallas_call(
        _tc_in_body,
        out_shape=(
            jax.ShapeDtypeStruct((N, H), _f32),
            jax.ShapeDtypeStruct((N, 1), _f32),
        ),
    )(x, w_in, b_in, w1, degp)


def _tc_mid_body(agg_ref, hwp_ref, invdeg_ref, b_ref, w_ref, out_ref):
    acc = (agg_ref[0] + agg_ref[1]
           + invdeg_ref[...] * hwp_ref[...] + b_ref[...][None, :])
    h = jnp.maximum(acc, 0.0)
    out_ref[...] = lax.dot_general(h, w_ref[...], _CN,
                                   preferred_element_type=_f32)


def _tc_mid(agg, hw_prev, invdeg, b, w_next):
    return pl.pallas_call(
        _tc_mid_body,
        out_shape=jax.ShapeDtypeStruct((N, H), _f32),
    )(agg, hw_prev, invdeg, b, w_next)


def _tc_out_body(agg_ref, hwp_ref, invdeg_ref, b_ref, wo_ref, bo_ref, out_ref):
    acc = (agg_ref[0] + agg_ref[1]
           + invdeg_ref[...] * hwp_ref[...] + b_ref[...][None, :])
    h = jnp.maximum(acc, 0.0)
    out_ref[...] = lax.dot_general(h, wo_ref[...], _CN,
                                   preferred_element_type=_f32) \
        + bo_ref[...][None, :]


def _tc_out(agg, hw_prev, invdeg, b, w_out, b_out):
    return pl.pallas_call(
        _tc_out_body,
        out_shape=jax.ShapeDtypeStruct((N, C), _f32),
    )(agg, hw_prev, invdeg, b, w_out, b_out)


def _layout(a, pad_value):
    n0 = NS * NB_C0 * BLK
    p0 = a[:n0].reshape(NS, NB_C0 * BLK)
    p0 = jnp.pad(p0, ((0, 0), (0, EWMAX - NB_C0 * BLK)),
                 constant_values=pad_value)
    p1 = a[n0:].reshape(NS, NB_C1 * BLK)
    p1 = jnp.pad(p1, ((0, 0), (0, EWMAX - NB_C1 * BLK)),
                 constant_values=pad_value)
    return jnp.concatenate([p0, p1], axis=0)


def kernel(x, edge_index, edge_weight, W_in, b_in, W1, b1, W2, b2, W3, b3,
           W_out, b_out):
    src = edge_index[0]
    dst = edge_index[1]
    pad = E_PAD - E
    zpad_i = jnp.zeros((pad,), _i32)
    srcp = _layout(jnp.concatenate([src, zpad_i]), 0)
    dstp = _layout(jnp.concatenate([dst, zpad_i]), 0)
    ewp = _layout(jnp.concatenate([edge_weight, jnp.zeros((pad,), _f32)]), 0)
    zrows = jnp.zeros((RPS, H), _f32)

    degp = _sc_deg(dstp, ewp)
    hw1, invdeg = _tc_in(x, W_in, b_in, W1, degp)
    normp = _sc_norm(srcp, dstp, ewp, degp)
    agg1 = _sc_agg(hw1, srcp, dstp, normp, zrows)
    hw2 = _tc_mid(agg1, hw1, invdeg, b1, W2)
    agg2 = _sc_agg(hw2, srcp, dstp, normp, zrows)
    hw3 = _tc_mid(agg2, hw2, invdeg, b2, W3)
    agg3 = _sc_agg(hw3, srcp, dstp, normp, zrows)
    return _tc_out(agg3, hw3, invdeg, b3, W_out, b_out)


# speedup vs baseline: 10.4637x; 1.0806x over previous
import functools

import jax
import jax.numpy as jnp
from jax import lax
from jax.experimental import pallas as pl
from jax.experimental.pallas import tpu as pltpu
from jax.experimental.pallas import tpu_sc as plsc

N = 10000
E = 320000
H = 128
C = 40

NC = 2
NS = 16
NW = NC * NS
L = 16

BLK = 128
NB_C0 = 97
NB_C1 = 61
NBMAX = max(NB_C0, NB_C1)
EWMAX = NBMAX * BLK
E_PAD = NS * (NB_C0 + NB_C1) * BLK
RPS = 624
NP = 10112

_f32 = jnp.float32
_i32 = jnp.int32
_bf16 = jnp.bfloat16

_mesh = plsc.VectorSubcoreMesh(
    core_axis_name="c", subcore_axis_name="s", num_cores=NC, num_subcores=NS)


def _wid():
    c = lax.axis_index("c")
    s = lax.axis_index("s")
    return c, s, c * NS + s


_SC_DEG_PARAMS = dict(
    out_type=jax.ShapeDtypeStruct((NC, NP), _f32),
    mesh=_mesh,
    compiler_params=pltpu.CompilerParams(needs_layout_passes=False),
    scratch_types=[
        pltpu.VMEM((EWMAX,), _i32),
        pltpu.VMEM((EWMAX,), _f32),
        pltpu.VMEM((NP,), _f32),
        pltpu.VMEM((512,), _f32),
        pltpu.VMEM((512,), _f32),
        pltpu.VMEM((128,), _f32),
        pltpu.VMEM((128,), _f32),
        pltpu.VMEM_SHARED((NS * NP,), _f32),
    ],
)


def _sc_deg_body(dstp_hbm, ewp_hbm, out_hbm, dstv, ewv, accv,
                 cb512, acc512, cb128, acc128, slab):
    c, s, w = _wid()
    pltpu.sync_copy(dstp_hbm.at[w], dstv)
    pltpu.sync_copy(ewp_hbm.at[w], ewv)

    def zbody(i, carry):
        accv[pl.ds(i * L, L)] = jnp.zeros((L,), _f32)
        return carry

    lax.fori_loop(0, NP // L, zbody, 0)

    def body(i, carry):
        sl = pl.ds(i * L, L)
        plsc.addupdate_scatter(accv, [dstv[sl]], ewv[sl])
        return carry

    lax.fori_loop(0, EWMAX // L, body, 0)
    pltpu.sync_copy(accv, slab.at[pl.ds(s * NP, NP)])
    plsc.subcore_barrier()

    for i in range(512 // L):
        acc512[pl.ds(i * L, L)] = jnp.zeros((L,), _f32)
    for i in range(128 // L):
        acc128[pl.ds(i * L, L)] = jnp.zeros((L,), _f32)

    def rbody(r, carry):
        pltpu.sync_copy(slab.at[pl.ds(r * NP + s * 640, 512)], cb512)

        def add5(i, carry2):
            sl = pl.ds(i * L, L)
            acc512[sl] = acc512[sl] + cb512[sl]
            return carry2

        lax.fori_loop(0, 512 // L, add5, 0)

        @pl.when(s < NS - 1)
        def _():
            pltpu.sync_copy(slab.at[pl.ds(r * NP + s * 640 + 512, 128)],
                            cb128)

            def add1(i, carry2):
                sl = pl.ds(i * L, L)
                acc128[sl] = acc128[sl] + cb128[sl]
                return carry2

            lax.fori_loop(0, 128 // L, add1, 0)

        return carry

    lax.fori_loop(0, NS, rbody, 0)
    pltpu.sync_copy(acc512, out_hbm.at[c, pl.ds(s * 640, 512)])

    @pl.when(s < NS - 1)
    def _():
        pltpu.sync_copy(acc128, out_hbm.at[c, pl.ds(s * 640 + 512, 128)])


def _rsqrt16(d):
    xi = plsc.bitcast(d, _i32)
    yi = jnp.int32(0x5F3759DF) - lax.shift_right_logical(xi, 1)
    y = plsc.bitcast(yi, _f32)
    for _ in range(3):
        y = y * (1.5 - 0.5 * d * y * y)
    return y


_SC_NORM_PARAMS = dict(
    out_type=jax.ShapeDtypeStruct((NW, EWMAX // 2), _i32),
    mesh=_mesh,
    compiler_params=pltpu.CompilerParams(needs_layout_passes=False),
    scratch_types=[
        pltpu.VMEM((EWMAX,), _i32),
        pltpu.VMEM((EWMAX,), _i32),
        pltpu.VMEM((EWMAX,), _f32),
        pltpu.VMEM((NP,), _f32),
        pltpu.VMEM((NP,), _f32),
        pltpu.VMEM((EWMAX // 2,), _i32),
    ],
)


def _sc_norm_body(srcp_hbm, dstp_hbm, ewp_hbm, degp_hbm, out_hbm,
                  srcv, dstv, ewv, disv, pbuf, normh):
    _, _, w = _wid()
    pltpu.sync_copy(srcp_hbm.at[w], srcv)
    pltpu.sync_copy(dstp_hbm.at[w], dstv)
    pltpu.sync_copy(ewp_hbm.at[w], ewv)

    def zbody(i, carry):
        disv[pl.ds(i * L, L)] = jnp.zeros((L,), _f32)
        return carry

    lax.fori_loop(0, NP // L, zbody, 0)

    def abody(i, carry):
        sl = pl.ds(i * L, L)
        disv[sl] = disv[sl] + pbuf[sl]
        return carry

    for r in range(NC):
        pltpu.sync_copy(degp_hbm.at[r], pbuf)
        lax.fori_loop(0, NP // L, abody, 0)

    def dbody(i, carry):
        sl = pl.ds(i * L, L)
        disv[sl] = _rsqrt16(disv[sl] + 1.0)
        return carry

    lax.fori_loop(0, NP // L, dbody, 0)

    def body(g, carry):
        ns = []
        for half in range(2):
            sl = pl.ds(g * 2 * L + half * L, L)
            a = plsc.load_gather(disv, [srcv[sl]])
            b = plsc.load_gather(disv, [dstv[sl]])
            ns.append(a * ewv[sl] * b)
        packed = plsc.pack(ns[0], ns[1], format=plsc.PackFormat.INTERLEAVED)
        normh[pl.ds(g * L, L)] = plsc.bitcast(packed, _i32)
        return carry

    lax.fori_loop(0, EWMAX // (2 * L), body, 0)
    pltpu.sync_copy(normh, out_hbm.at[w])


_SC_AGG_PARAMS = dict(
    out_type=jax.ShapeDtypeStruct((NC, N, H), _f32),
    mesh=_mesh,
    compiler_params=pltpu.CompilerParams(needs_layout_passes=False),
    scratch_types=[
        pltpu.VMEM((EWMAX,), _i32),
        pltpu.VMEM((EWMAX,), _i32),
        pltpu.VMEM((EWMAX // 2,), _i32),
        pltpu.VMEM((BLK, H), _f32),
        pltpu.SemaphoreType.DMA,
        pltpu.SemaphoreType.DMA,
        pltpu.VMEM_SHARED((N, H), _f32),
    ],
)


def _sc_agg_body(hw_hbm, srcp_hbm, dstp_hbm, normp_hbm, zrows_hbm, out_hbm,
                 srcv, dstv, normh, gbuf, gsem, ssem, acc):
    c, s, w = _wid()
    pltpu.sync_copy(srcp_hbm.at[w], srcv)
    pltpu.sync_copy(dstp_hbm.at[w], dstv)
    pltpu.sync_copy(normp_hbm.at[w], normh)
    pltpu.sync_copy(zrows_hbm, acc.at[pl.ds(s * RPS, RPS)])

    @pl.when(s == NS - 1)
    def _():
        pltpu.sync_copy(zrows_hbm.at[pl.ds(0, N - NS * RPS)],
                        acc.at[pl.ds(NS * RPS, N - NS * RPS)])

    plsc.subcore_barrier()

    nblocks = jnp.where(c == 0, NB_C0, NB_C1)

    def blk_body(bi, carry):
        idx = srcv.at[pl.ds(bi * BLK, BLK)]
        pltpu.async_copy(hw_hbm.at[idx], gbuf, gsem).wait()

        def g_body(g, carry2):
            nh32 = normh[pl.ds(bi * (BLK // 2) + g * L, L)]
            nh = plsc.bitcast(nh32, _bf16)
            na, nb = plsc.unpack(nh, format=plsc.PackFormat.INTERLEAVED)
            for half, nv in ((0, na), (1, nb)):
                for k in range(L):
                    n = nv[k]
                    e = g * 2 * L + half * L + k
                    for j in range(H // L):
                        sl = pl.ds(j * L, L)
                        gbuf[e, sl] = gbuf[e, sl] * n
            return carry2

        lax.fori_loop(0, BLK // (2 * L), g_body, 0)
        didx = dstv.at[pl.ds(bi * BLK, BLK)]
        pltpu.async_copy(gbuf, acc.at[didx], ssem, add=True).wait()
        return carry

    lax.fori_loop(0, nblocks, blk_body, 0)
    plsc.subcore_barrier()
    sl = pl.ds(s * RPS, RPS)
    pltpu.sync_copy(acc.at[sl], out_hbm.at[c, sl])

    @pl.when(s == NS - 1)
    def _():
        sl2 = pl.ds(NS * RPS, N - NS * RPS)
        pltpu.sync_copy(acc.at[sl2], out_hbm.at[c, sl2])


_sc_deg = pl.kernel(_sc_deg_body, **_SC_DEG_PARAMS)
_sc_norm = pl.kernel(_sc_norm_body, **_SC_NORM_PARAMS)
_sc_agg = pl.kernel(_sc_agg_body, **_SC_AGG_PARAMS)


_CN = (((1,), (1,)), ((), ()))


def _tc_in_body(x_ref, wi_ref, bi_ref, w1_ref, degp_ref,
                hw1_ref, invdeg_ref):
    h = lax.dot_general(x_ref[...], wi_ref[...], _CN,
                        preferred_element_type=_f32)
    h = jnp.maximum(h + bi_ref[...][None, :], 0.0)
    hw1_ref[...] = lax.dot_general(h, w1_ref[...], _CN,
                                   preferred_element_type=_f32)
    ones = jnp.ones((NC, 1), _f32)
    deg2 = lax.dot_general(degp_ref[...], ones, (((0,), (0,)), ((), ())),
                           preferred_element_type=_f32) + 1.0
    invdeg_ref[...] = 1.0 / deg2[:N, :]


def _tc_in(x, w_in, b_in, w1, degp):
    return pl.pallas_call(
        _tc_in_body,
        out_shape=(
            jax.ShapeDtypeStruct((N, H), _f32),
            jax.ShapeDtypeStruct((N, 1), _f32),
        ),
    )(x, w_in, b_in, w1, degp)


def _tc_mid_body(agg_ref, hwp_ref, invdeg_ref, b_ref, w_ref, out_ref):
    acc = (agg_ref[0] + agg_ref[1]
           + invdeg_ref[...] * hwp_ref[...] + b_ref[...][None, :])
    h = jnp.maximum(acc, 0.0)
    out_ref[...] = lax.dot_general(h, w_ref[...], _CN,
                                   preferred_element_type=_f32)


def _tc_mid(agg, hw_prev, invdeg, b, w_next):
    return pl.pallas_call(
        _tc_mid_body,
        out_shape=jax.ShapeDtypeStruct((N, H), _f32),
    )(agg, hw_prev, invdeg, b, w_next)


def _tc_out_body(agg_ref, hwp_ref, invdeg_ref, b_ref, wo_ref, bo_ref, out_ref):
    acc = (agg_ref[0] + agg_ref[1]
           + invdeg_ref[...] * hwp_ref[...] + b_ref[...][None, :])
    h = jnp.maximum(acc, 0.0)
    out_ref[...] = lax.dot_general(h, wo_ref[...], _CN,
                                   preferred_element_type=_f32) \
        + bo_ref[...][None, :]


def _tc_out(agg, hw_prev, invdeg, b, w_out, b_out):
    return pl.pallas_call(
        _tc_out_body,
        out_shape=jax.ShapeDtypeStruct((N, C), _f32),
    )(agg, hw_prev, invdeg, b, w_out, b_out)


def _layout(a, pad_value):
    n0 = NS * NB_C0 * BLK
    p0 = a[:n0].reshape(NS, NB_C0 * BLK)
    p0 = jnp.pad(p0, ((0, 0), (0, EWMAX - NB_C0 * BLK)),
                 constant_values=pad_value)
    p1 = a[n0:].reshape(NS, NB_C1 * BLK)
    p1 = jnp.pad(p1, ((0, 0), (0, EWMAX - NB_C1 * BLK)),
                 constant_values=pad_value)
    return jnp.concatenate([p0, p1], axis=0)


def kernel(x, edge_index, edge_weight, W_in, b_in, W1, b1, W2, b2, W3, b3,
           W_out, b_out):
    src = edge_index[0]
    dst = edge_index[1]
    pad = E_PAD - E
    zpad_i = jnp.zeros((pad,), _i32)
    srcp = _layout(jnp.concatenate([src, zpad_i]), 0)
    dstp = _layout(jnp.concatenate([dst, zpad_i]), 0)
    ewp = _layout(jnp.concatenate([edge_weight, jnp.zeros((pad,), _f32)]), 0)
    zrows = jnp.zeros((RPS, H), _f32)

    degp = _sc_deg(dstp, ewp)
    hw1, invdeg = _tc_in(x, W_in, b_in, W1, degp)
    normp = _sc_norm(srcp, dstp, ewp, degp)
    agg1 = _sc_agg(hw1, srcp, dstp, normp, zrows)
    hw2 = _tc_mid(agg1, hw1, invdeg, b1, W2)
    agg2 = _sc_agg(hw2, srcp, dstp, normp, zrows)
    hw3 = _tc_mid(agg2, hw2, invdeg, b2, W3)
    agg3 = _sc_agg(hw3, srcp, dstp, normp, zrows)
    return _tc_out(agg3, hw3, invdeg, b3, W_out, b_out)


# speedup vs baseline: 11.3682x; 1.0864x over previous
import functools

import jax
import jax.numpy as jnp
from jax import lax
from jax.experimental import pallas as pl
from jax.experimental.pallas import tpu as pltpu
from jax.experimental.pallas import tpu_sc as plsc

N = 10000
E = 320000
H = 128
C = 40

NC = 2
NS = 16
NW = NC * NS
L = 16

BLK = 128
NB_C0 = 104
NB_C1 = 54
NBMAX = max(NB_C0, NB_C1)
EWMAX = NBMAX * BLK
E_PAD = NS * (NB_C0 + NB_C1) * BLK
RPS = 624
NP = 10112

_f32 = jnp.float32
_i32 = jnp.int32
_bf16 = jnp.bfloat16

_mesh = plsc.VectorSubcoreMesh(
    core_axis_name="c", subcore_axis_name="s", num_cores=NC, num_subcores=NS)


def _wid():
    c = lax.axis_index("c")
    s = lax.axis_index("s")
    return c, s, c * NS + s


_SC_DEG_PARAMS = dict(
    out_type=jax.ShapeDtypeStruct((NC, NP), _f32),
    mesh=_mesh,
    compiler_params=pltpu.CompilerParams(needs_layout_passes=False),
    scratch_types=[
        pltpu.VMEM((EWMAX,), _i32),
        pltpu.VMEM((EWMAX,), _f32),
        pltpu.VMEM((NP,), _f32),
        pltpu.VMEM((512,), _f32),
        pltpu.VMEM((512,), _f32),
        pltpu.VMEM((128,), _f32),
        pltpu.VMEM((128,), _f32),
        pltpu.VMEM_SHARED((NS * NP,), _f32),
    ],
)


def _sc_deg_body(dstp_hbm, ewp_hbm, out_hbm, dstv, ewv, accv,
                 cb512, acc512, cb128, acc128, slab):
    c, s, w = _wid()
    pltpu.sync_copy(dstp_hbm.at[w], dstv)
    pltpu.sync_copy(ewp_hbm.at[w], ewv)

    def zbody(i, carry):
        accv[pl.ds(i * L, L)] = jnp.zeros((L,), _f32)
        return carry

    lax.fori_loop(0, NP // L, zbody, 0)

    def body(i, carry):
        sl = pl.ds(i * L, L)
        plsc.addupdate_scatter(accv, [dstv[sl]], ewv[sl])
        return carry

    lax.fori_loop(0, EWMAX // L, body, 0)
    pltpu.sync_copy(accv, slab.at[pl.ds(s * NP, NP)])
    plsc.subcore_barrier()

    for i in range(512 // L):
        acc512[pl.ds(i * L, L)] = jnp.zeros((L,), _f32)
    for i in range(128 // L):
        acc128[pl.ds(i * L, L)] = jnp.zeros((L,), _f32)

    def rbody(r, carry):
        pltpu.sync_copy(slab.at[pl.ds(r * NP + s * 640, 512)], cb512)

        def add5(i, carry2):
            sl = pl.ds(i * L, L)
            acc512[sl] = acc512[sl] + cb512[sl]
            return carry2

        lax.fori_loop(0, 512 // L, add5, 0)

        @pl.when(s < NS - 1)
        def _():
            pltpu.sync_copy(slab.at[pl.ds(r * NP + s * 640 + 512, 128)],
                            cb128)

            def add1(i, carry2):
                sl = pl.ds(i * L, L)
                acc128[sl] = acc128[sl] + cb128[sl]
                return carry2

            lax.fori_loop(0, 128 // L, add1, 0)

        return carry

    lax.fori_loop(0, NS, rbody, 0)
    pltpu.sync_copy(acc512, out_hbm.at[c, pl.ds(s * 640, 512)])

    @pl.when(s < NS - 1)
    def _():
        pltpu.sync_copy(acc128, out_hbm.at[c, pl.ds(s * 640 + 512, 128)])


def _rsqrt16(d):
    xi = plsc.bitcast(d, _i32)
    yi = jnp.int32(0x5F3759DF) - lax.shift_right_logical(xi, 1)
    y = plsc.bitcast(yi, _f32)
    for _ in range(3):
        y = y * (1.5 - 0.5 * d * y * y)
    return y


_SC_NORM_PARAMS = dict(
    out_type=jax.ShapeDtypeStruct((NW, EWMAX // 2), _i32),
    mesh=_mesh,
    compiler_params=pltpu.CompilerParams(needs_layout_passes=False),
    scratch_types=[
        pltpu.VMEM((EWMAX,), _i32),
        pltpu.VMEM((EWMAX,), _i32),
        pltpu.VMEM((EWMAX,), _f32),
        pltpu.VMEM((NP,), _f32),
        pltpu.VMEM((NP,), _f32),
        pltpu.VMEM((EWMAX // 2,), _i32),
    ],
)


def _sc_norm_body(srcp_hbm, dstp_hbm, ewp_hbm, degp_hbm, out_hbm,
                  srcv, dstv, ewv, disv, pbuf, normh):
    _, _, w = _wid()
    pltpu.sync_copy(srcp_hbm.at[w], srcv)
    pltpu.sync_copy(dstp_hbm.at[w], dstv)
    pltpu.sync_copy(ewp_hbm.at[w], ewv)

    def zbody(i, carry):
        disv[pl.ds(i * L, L)] = jnp.zeros((L,), _f32)
        return carry

    lax.fori_loop(0, NP // L, zbody, 0)

    def abody(i, carry):
        sl = pl.ds(i * L, L)
        disv[sl] = disv[sl] + pbuf[sl]
        return carry

    for r in range(NC):
        pltpu.sync_copy(degp_hbm.at[r], pbuf)
        lax.fori_loop(0, NP // L, abody, 0)

    def dbody(i, carry):
        sl = pl.ds(i * L, L)
        disv[sl] = _rsqrt16(disv[sl] + 1.0)
        return carry

    lax.fori_loop(0, NP // L, dbody, 0)

    def body(g, carry):
        ns = []
        for half in range(2):
            sl = pl.ds(g * 2 * L + half * L, L)
            a = plsc.load_gather(disv, [srcv[sl]])
            b = plsc.load_gather(disv, [dstv[sl]])
            ns.append(a * ewv[sl] * b)
        packed = plsc.pack(ns[0], ns[1], format=plsc.PackFormat.INTERLEAVED)
        normh[pl.ds(g * L, L)] = plsc.bitcast(packed, _i32)
        return carry

    lax.fori_loop(0, EWMAX // (2 * L), body, 0)
    pltpu.sync_copy(normh, out_hbm.at[w])


_SC_AGG_PARAMS = dict(
    out_type=jax.ShapeDtypeStruct((NC, N, H), _f32),
    mesh=_mesh,
    compiler_params=pltpu.CompilerParams(needs_layout_passes=False),
    scratch_types=[
        pltpu.VMEM((EWMAX,), _i32),
        pltpu.VMEM((EWMAX,), _i32),
        pltpu.VMEM((EWMAX // 2,), _i32),
        pltpu.VMEM((BLK, H), _f32),
        pltpu.SemaphoreType.DMA,
        pltpu.SemaphoreType.DMA,
        pltpu.VMEM_SHARED((N, H), _f32),
    ],
)


def _sc_agg_body(hw_hbm, srcp_hbm, dstp_hbm, normp_hbm, zrows_hbm, out_hbm,
                 srcv, dstv, normh, gbuf, gsem, ssem, acc):
    c, s, w = _wid()
    pltpu.sync_copy(srcp_hbm.at[w], srcv)
    pltpu.sync_copy(dstp_hbm.at[w], dstv)
    pltpu.sync_copy(normp_hbm.at[w], normh)
    pltpu.sync_copy(zrows_hbm, acc.at[pl.ds(s * RPS, RPS)])

    @pl.when(s == NS - 1)
    def _():
        pltpu.sync_copy(zrows_hbm.at[pl.ds(0, N - NS * RPS)],
                        acc.at[pl.ds(NS * RPS, N - NS * RPS)])

    plsc.subcore_barrier()

    nblocks = jnp.where(c == 0, NB_C0, NB_C1)

    def blk_body(bi, carry):
        idx = srcv.at[pl.ds(bi * BLK, BLK)]
        pltpu.async_copy(hw_hbm.at[idx], gbuf, gsem).wait()

        def g_body(g, carry2):
            nh32 = normh[pl.ds(bi * (BLK // 2) + g * L, L)]
            nh = plsc.bitcast(nh32, _bf16)
            na, nb = plsc.unpack(nh, format=plsc.PackFormat.INTERLEAVED)
            for half, nv in ((0, na), (1, nb)):
                for k in range(L):
                    n = nv[k]
                    e = g * 2 * L + half * L + k
                    for j in range(H // L):
                        sl = pl.ds(j * L, L)
                        gbuf[e, sl] = gbuf[e, sl] * n
            return carry2

        lax.fori_loop(0, BLK // (2 * L), g_body, 0)
        didx = dstv.at[pl.ds(bi * BLK, BLK)]
        pltpu.async_copy(gbuf, acc.at[didx], ssem, add=True).wait()
        return carry

    lax.fori_loop(0, nblocks, blk_body, 0)
    plsc.subcore_barrier()
    sl = pl.ds(s * RPS, RPS)
    pltpu.sync_copy(acc.at[sl], out_hbm.at[c, sl])

    @pl.when(s == NS - 1)
    def _():
        sl2 = pl.ds(NS * RPS, N - NS * RPS)
        pltpu.sync_copy(acc.at[sl2], out_hbm.at[c, sl2])


_sc_deg = pl.kernel(_sc_deg_body, **_SC_DEG_PARAMS)
_sc_norm = pl.kernel(_sc_norm_body, **_SC_NORM_PARAMS)
_sc_agg = pl.kernel(_sc_agg_body, **_SC_AGG_PARAMS)


_CN = (((1,), (1,)), ((), ()))


def _tc_in_body(x_ref, wi_ref, bi_ref, w1_ref, degp_ref,
                hw1_ref, invdeg_ref):
    h = lax.dot_general(x_ref[...], wi_ref[...], _CN,
                        preferred_element_type=_f32)
    h = jnp.maximum(h + bi_ref[...][None, :], 0.0)
    hw1_ref[...] = lax.dot_general(h, w1_ref[...], _CN,
                                   preferred_element_type=_f32)
    ones = jnp.ones((NC, 1), _f32)
    deg2 = lax.dot_general(degp_ref[...], ones, (((0,), (0,)), ((), ())),
                           preferred_element_type=_f32) + 1.0
    invdeg_ref[...] = 1.0 / deg2[:N, :]


def _tc_in(x, w_in, b_in, w1, degp):
    return pl.pallas_call(
        _tc_in_body,
        out_shape=(
            jax.ShapeDtypeStruct((N, H), _f32),
            jax.ShapeDtypeStruct((N, 1), _f32),
        ),
    )(x, w_in, b_in, w1, degp)


def _tc_mid_body(agg_ref, hwp_ref, invdeg_ref, b_ref, w_ref, out_ref):
    acc = (agg_ref[0] + agg_ref[1]
           + invdeg_ref[...] * hwp_ref[...] + b_ref[...][None, :])
    h = jnp.maximum(acc, 0.0)
    out_ref[...] = lax.dot_general(h, w_ref[...], _CN,
                                   preferred_element_type=_f32)


def _tc_mid(agg, hw_prev, invdeg, b, w_next):
    return pl.pallas_call(
        _tc_mid_body,
        out_shape=jax.ShapeDtypeStruct((N, H), _f32),
    )(agg, hw_prev, invdeg, b, w_next)


def _tc_out_body(agg_ref, hwp_ref, invdeg_ref, b_ref, wo_ref, bo_ref, out_ref):
    acc = (agg_ref[0] + agg_ref[1]
           + invdeg_ref[...] * hwp_ref[...] + b_ref[...][None, :])
    h = jnp.maximum(acc, 0.0)
    out_ref[...] = lax.dot_general(h, wo_ref[...], _CN,
                                   preferred_element_type=_f32) \
        + bo_ref[...][None, :]


def _tc_out(agg, hw_prev, invdeg, b, w_out, b_out):
    return pl.pallas_call(
        _tc_out_body,
        out_shape=jax.ShapeDtypeStruct((N, C), _f32),
    )(agg, hw_prev, invdeg, b, w_out, b_out)


def _layout(a, pad_value):
    n0 = NS * NB_C0 * BLK
    p0 = a[:n0].reshape(NS, NB_C0 * BLK)
    p0 = jnp.pad(p0, ((0, 0), (0, EWMAX - NB_C0 * BLK)),
                 constant_values=pad_value)
    p1 = a[n0:].reshape(NS, NB_C1 * BLK)
    p1 = jnp.pad(p1, ((0, 0), (0, EWMAX - NB_C1 * BLK)),
                 constant_values=pad_value)
    return jnp.concatenate([p0, p1], axis=0)


def kernel(x, edge_index, edge_weight, W_in, b_in, W1, b1, W2, b2, W3, b3,
           W_out, b_out):
    src = edge_index[0]
    dst = edge_index[1]
    pad = E_PAD - E
    zpad_i = jnp.zeros((pad,), _i32)
    srcp = _layout(jnp.concatenate([src, zpad_i]), 0)
    dstp = _layout(jnp.concatenate([dst, zpad_i]), 0)
    ewp = _layout(jnp.concatenate([edge_weight, jnp.zeros((pad,), _f32)]), 0)
    zrows = jnp.zeros((RPS, H), _f32)

    degp = _sc_deg(dstp, ewp)
    hw1, invdeg = _tc_in(x, W_in, b_in, W1, degp)
    normp = _sc_norm(srcp, dstp, ewp, degp)
    agg1 = _sc_agg(hw1, srcp, dstp, normp, zrows)
    hw2 = _tc_mid(agg1, hw1, invdeg, b1, W2)
    agg2 = _sc_agg(hw2, srcp, dstp, normp, zrows)
    hw3 = _tc_mid(agg2, hw2, invdeg, b2, W3)
    agg3 = _sc_agg(hw3, srcp, dstp, normp, zrows)
    return _tc_out(agg3, hw3, invdeg, b3, W_out, b_out)


# speedup vs baseline: 11.3702x; 1.0002x over previous
import jax
import jax.numpy as jnp
from jax import lax
from jax.experimental import pallas as pl
from jax.experimental.pallas import tpu as pltpu
from jax.experimental.pallas import tpu_sc as plsc

N = 10000
E = 320000
H = 128
C = 40

NC = 2
NS = 16
NW = NC * NS
L = 16

BLK = 128
NB_C0 = 104
NB_C1 = 54
NBMAX = max(NB_C0, NB_C1)
EWMAX = NBMAX * BLK
E_PAD = NS * (NB_C0 + NB_C1) * BLK
RPS = 624
NP = 10112

_f32 = jnp.float32
_i32 = jnp.int32
_bf16 = jnp.bfloat16

_mesh = plsc.VectorSubcoreMesh(
    core_axis_name="c", subcore_axis_name="s", num_cores=NC, num_subcores=NS)


def _wid():
    c = lax.axis_index("c")
    s = lax.axis_index("s")
    return c, s, c * NS + s


_SC_DEG_PARAMS = dict(
    out_type=jax.ShapeDtypeStruct((NC, NP), _f32),
    mesh=_mesh,
    compiler_params=pltpu.CompilerParams(needs_layout_passes=False),
    scratch_types=[
        pltpu.VMEM((EWMAX,), _i32),
        pltpu.VMEM((EWMAX,), _f32),
        pltpu.VMEM((NP,), _f32),
        pltpu.VMEM((512,), _f32),
        pltpu.VMEM((512,), _f32),
        pltpu.VMEM((128,), _f32),
        pltpu.VMEM((128,), _f32),
        pltpu.VMEM_SHARED((NS * NP,), _f32),
    ],
)


def _sc_deg_body(dstp_hbm, ewp_hbm, out_hbm, dstv, ewv, accv,
                 cb512, acc512, cb128, acc128, slab):
    c, s, w = _wid()
    pltpu.sync_copy(dstp_hbm.at[w], dstv)
    pltpu.sync_copy(ewp_hbm.at[w], ewv)

    def zbody(i, carry):
        accv[pl.ds(i * L, L)] = jnp.zeros((L,), _f32)
        return carry

    lax.fori_loop(0, NP // L, zbody, 0)

    def body(i, carry):
        sl = pl.ds(i * L, L)
        plsc.addupdate_scatter(accv, [dstv[sl]], ewv[sl])
        return carry

    lax.fori_loop(0, EWMAX // L, body, 0)
    pltpu.sync_copy(accv, slab.at[pl.ds(s * NP, NP)])
    plsc.subcore_barrier()

    for i in range(512 // L):
        acc512[pl.ds(i * L, L)] = jnp.zeros((L,), _f32)
    for i in range(128 // L):
        acc128[pl.ds(i * L, L)] = jnp.zeros((L,), _f32)

    def rbody(r, carry):
        pltpu.sync_copy(slab.at[pl.ds(r * NP + s * 640, 512)], cb512)

        def add5(i, carry2):
            sl = pl.ds(i * L, L)
            acc512[sl] = acc512[sl] + cb512[sl]
            return carry2

        lax.fori_loop(0, 512 // L, add5, 0)

        @pl.when(s < NS - 1)
        def _():
            pltpu.sync_copy(slab.at[pl.ds(r * NP + s * 640 + 512, 128)],
                            cb128)

            def add1(i, carry2):
                sl = pl.ds(i * L, L)
                acc128[sl] = acc128[sl] + cb128[sl]
                return carry2

            lax.fori_loop(0, 128 // L, add1, 0)

        return carry

    lax.fori_loop(0, NS, rbody, 0)
    pltpu.sync_copy(acc512, out_hbm.at[c, pl.ds(s * 640, 512)])

    @pl.when(s < NS - 1)
    def _():
        pltpu.sync_copy(acc128, out_hbm.at[c, pl.ds(s * 640 + 512, 128)])


def _rsqrt16(d):
    xi = plsc.bitcast(d, _i32)
    yi = jnp.int32(0x5F3759DF) - lax.shift_right_logical(xi, 1)
    y = plsc.bitcast(yi, _f32)
    for _ in range(3):
        y = y * (1.5 - 0.5 * d * y * y)
    return y


_SC_NORM_PARAMS = dict(
    out_type=jax.ShapeDtypeStruct((NW, EWMAX // 2), _i32),
    mesh=_mesh,
    compiler_params=pltpu.CompilerParams(needs_layout_passes=False),
    scratch_types=[
        pltpu.VMEM((EWMAX,), _i32),
        pltpu.VMEM((EWMAX,), _i32),
        pltpu.VMEM((EWMAX,), _f32),
        pltpu.VMEM((NP,), _f32),
        pltpu.VMEM((NP,), _f32),
        pltpu.VMEM((EWMAX // 2,), _i32),
    ],
)


def _sc_norm_body(srcp_hbm, dstp_hbm, ewp_hbm, degp_hbm, out_hbm,
                  srcv, dstv, ewv, disv, pbuf, normh):
    _, _, w = _wid()
    pltpu.sync_copy(srcp_hbm.at[w], srcv)
    pltpu.sync_copy(dstp_hbm.at[w], dstv)
    pltpu.sync_copy(ewp_hbm.at[w], ewv)

    def zbody(i, carry):
        disv[pl.ds(i * L, L)] = jnp.zeros((L,), _f32)
        return carry

    lax.fori_loop(0, NP // L, zbody, 0)

    def abody(i, carry):
        sl = pl.ds(i * L, L)
        disv[sl] = disv[sl] + pbuf[sl]
        return carry

    for r in range(NC):
        pltpu.sync_copy(degp_hbm.at[r], pbuf)
        lax.fori_loop(0, NP // L, abody, 0)

    def dbody(i, carry):
        sl = pl.ds(i * L, L)
        disv[sl] = _rsqrt16(disv[sl] + 1.0)
        return carry

    lax.fori_loop(0, NP // L, dbody, 0)

    def body(g, carry):
        ns = []
        for half in range(2):
            sl = pl.ds(g * 2 * L + half * L, L)
            a = plsc.load_gather(disv, [srcv[sl]])
            b = plsc.load_gather(disv, [dstv[sl]])
            ns.append(a * ewv[sl] * b)
        packed = plsc.pack(ns[0], ns[1], format=plsc.PackFormat.INTERLEAVED)
        normh[pl.ds(g * L, L)] = plsc.bitcast(packed, _i32)
        return carry

    lax.fori_loop(0, EWMAX // (2 * L), body, 0)
    pltpu.sync_copy(normh, out_hbm.at[w])


_SC_AGG_PARAMS = dict(
    out_type=jax.ShapeDtypeStruct((NC, N, H), _f32),
    mesh=_mesh,
    compiler_params=pltpu.CompilerParams(needs_layout_passes=False),
    scratch_types=[
        pltpu.VMEM((EWMAX,), _i32),
        pltpu.VMEM((EWMAX,), _i32),
        pltpu.VMEM((EWMAX // 2,), _i32),
        pltpu.VMEM((BLK, H), _f32),
        pltpu.SemaphoreType.DMA,
        pltpu.SemaphoreType.DMA,
        pltpu.VMEM_SHARED((N, H), _f32),
    ],
)


def _sc_agg_body(hw_hbm, srcp_hbm, dstp_hbm, normp_hbm, zrows_hbm, out_hbm,
                 srcv, dstv, normh, gbuf, gsem, ssem, acc):
    c, s, w = _wid()
    pltpu.sync_copy(srcp_hbm.at[w], srcv)
    pltpu.sync_copy(dstp_hbm.at[w], dstv)
    pltpu.sync_copy(normp_hbm.at[w], normh)
    pltpu.sync_copy(zrows_hbm, acc.at[pl.ds(s * RPS, RPS)])

    @pl.when(s == NS - 1)
    def _():
        pltpu.sync_copy(zrows_hbm.at[pl.ds(0, N - NS * RPS)],
                        acc.at[pl.ds(NS * RPS, N - NS * RPS)])

    plsc.subcore_barrier()

    nblocks = jnp.where(c == 0, NB_C0, NB_C1)

    def blk_body(bi, carry):
        idx = srcv.at[pl.ds(bi * BLK, BLK)]
        pltpu.async_copy(hw_hbm.at[idx], gbuf, gsem).wait()

        def g_body(g, carry2):
            nh32 = normh[pl.ds(bi * (BLK // 2) + g * L, L)]
            nh = plsc.bitcast(nh32, _bf16)
            na, nb = plsc.unpack(nh, format=plsc.PackFormat.INTERLEAVED)
            for half, nv in ((0, na), (1, nb)):
                for k in range(L):
                    n = nv[k]
                    e = g * 2 * L + half * L + k
                    for j in range(H // L):
                        sl = pl.ds(j * L, L)
                        gbuf[e, sl] = gbuf[e, sl] * n
            return carry2

        lax.fori_loop(0, BLK // (2 * L), g_body, 0)
        didx = dstv.at[pl.ds(bi * BLK, BLK)]
        pltpu.async_copy(gbuf, acc.at[didx], ssem, add=True).wait()
        return carry

    lax.fori_loop(0, nblocks, blk_body, 0)
    plsc.subcore_barrier()
    sl = pl.ds(s * RPS, RPS)
    pltpu.sync_copy(acc.at[sl], out_hbm.at[c, sl])

    @pl.when(s == NS - 1)
    def _():
        sl2 = pl.ds(NS * RPS, N - NS * RPS)
        pltpu.sync_copy(acc.at[sl2], out_hbm.at[c, sl2])


_sc_deg = pl.kernel(_sc_deg_body, **_SC_DEG_PARAMS)
_sc_norm = pl.kernel(_sc_norm_body, **_SC_NORM_PARAMS)
_sc_agg = pl.kernel(_sc_agg_body, **_SC_AGG_PARAMS)


_CN = (((1,), (1,)), ((), ()))


def _tc_in_body(x_ref, wi_ref, bi_ref, w1_ref, degp_ref,
                hw1_ref, invdeg_ref):
    h = lax.dot_general(x_ref[...], wi_ref[...], _CN,
                        preferred_element_type=_f32)
    h = jnp.maximum(h + bi_ref[...][None, :], 0.0)
    hw1_ref[...] = lax.dot_general(h, w1_ref[...], _CN,
                                   preferred_element_type=_f32)
    ones = jnp.ones((NC, 1), _f32)
    deg2 = lax.dot_general(degp_ref[...], ones, (((0,), (0,)), ((), ())),
                           preferred_element_type=_f32) + 1.0
    invdeg_ref[...] = 1.0 / deg2[:N, :]


def _tc_in(x, w_in, b_in, w1, degp):
    return pl.pallas_call(
        _tc_in_body,
        out_shape=(
            jax.ShapeDtypeStruct((N, H), _f32),
            jax.ShapeDtypeStruct((N, 1), _f32),
        ),
    )(x, w_in, b_in, w1, degp)


def _tc_mid_body(agg_ref, hwp_ref, invdeg_ref, b_ref, w_ref, out_ref):
    acc = (agg_ref[0] + agg_ref[1]
           + invdeg_ref[...] * hwp_ref[...] + b_ref[...][None, :])
    h = jnp.maximum(acc, 0.0)
    out_ref[...] = lax.dot_general(h, w_ref[...], _CN,
                                   preferred_element_type=_f32)


def _tc_mid(agg, hw_prev, invdeg, b, w_next):
    return pl.pallas_call(
        _tc_mid_body,
        out_shape=jax.ShapeDtypeStruct((N, H), _f32),
    )(agg, hw_prev, invdeg, b, w_next)


def _tc_out_body(agg_ref, hwp_ref, invdeg_ref, b_ref, wo_ref, bo_ref, out_ref):
    acc = (agg_ref[0] + agg_ref[1]
           + invdeg_ref[...] * hwp_ref[...] + b_ref[...][None, :])
    h = jnp.maximum(acc, 0.0)
    out_ref[...] = lax.dot_general(h, wo_ref[...], _CN,
                                   preferred_element_type=_f32) \
        + bo_ref[...][None, :]


def _tc_out(agg, hw_prev, invdeg, b, w_out, b_out):
    return pl.pallas_call(
        _tc_out_body,
        out_shape=jax.ShapeDtypeStruct((N, C), _f32),
    )(agg, hw_prev, invdeg, b, w_out, b_out)


def _layout(a, pad_value):
    n0 = NS * NB_C0 * BLK
    p0 = a[:n0].reshape(NS, NB_C0 * BLK)
    p0 = jnp.pad(p0, ((0, 0), (0, EWMAX - NB_C0 * BLK)),
                 constant_values=pad_value)
    p1 = a[n0:].reshape(NS, NB_C1 * BLK)
    p1 = jnp.pad(p1, ((0, 0), (0, EWMAX - NB_C1 * BLK)),
                 constant_values=pad_value)
    return jnp.concatenate([p0, p1], axis=0)


def kernel(x, edge_index, edge_weight, W_in, b_in, W1, b1, W2, b2, W3, b3,
           W_out, b_out):
    src = edge_index[0]
    dst = edge_index[1]
    pad = E_PAD - E
    zpad_i = jnp.zeros((pad,), _i32)
    srcp = _layout(jnp.concatenate([src, zpad_i]), 0)
    dstp = _layout(jnp.concatenate([dst, zpad_i]), 0)
    ewp = _layout(jnp.concatenate([edge_weight, jnp.zeros((pad,), _f32)]), 0)
    zrows = jnp.zeros((RPS, H), _f32)

    degp = _sc_deg(dstp, ewp)
    hw1, invdeg = _tc_in(x, W_in, b_in, W1, degp)
    normp = _sc_norm(srcp, dstp, ewp, degp)
    agg1 = _sc_agg(hw1, srcp, dstp, normp, zrows)
    hw2 = _tc_mid(agg1, hw1, invdeg, b1, W2)
    agg2 = _sc_agg(hw2, srcp, dstp, normp, zrows)
    hw3 = _tc_mid(agg2, hw2, invdeg, b2, W3)
    agg3 = _sc_agg(hw3, srcp, dstp, normp, zrows)
    return _tc_out(agg3, hw3, invdeg, b3, W_out, b_out)
